# Initial kernel scaffold; baseline (speedup 1.0000x reference)
import functools

import jax
import jax.numpy as jnp
from jax import lax
from jax.experimental import pallas as pl
from jax.experimental.pallas import tpu as pltpu
from jax.experimental.pallas import tpu_sc as plsc

N_NODES = 10000
D = 256
DH = 128

NC = 2
NS = 16
K = 128

E_PAD = 163840
EPT = E_PAD // NS
NCHUNK = EPT // K
ROWS_PT = N_NODES // NS


def _sc_aggregate(xs, col_r, row_r, ew_r):
  mesh = plsc.VectorSubcoreMesh(core_axis_name="c", subcore_axis_name="s")

  @functools.partial(
      pl.kernel,
      out_type=jax.ShapeDtypeStruct((N_NODES, D), jnp.float32),
      mesh=mesh,
      scratch_types=[
          pltpu.VMEM((NCHUNK, K), jnp.int32),
          pltpu.VMEM((NCHUNK, K), jnp.int32),
          pltpu.VMEM((NCHUNK, K), jnp.float32),
          pltpu.VMEM((K, DH), jnp.float32),
          pltpu.VMEM_SHARED((N_NODES, DH), jnp.float32),
      ],
  )
  def agg_kernel(xs_hbm, col_hbm, row_hbm, ew_hbm, out_hbm,
                 colv, rowv, wv, buf, acc):
    c = lax.axis_index("c")
    s = lax.axis_index("s")

    pltpu.sync_copy(col_hbm.at[s], colv)
    pltpu.sync_copy(row_hbm.at[s], rowv)
    pltpu.sync_copy(ew_hbm.at[s], wv)

    def zrow(i, _):
      for j in range(DH // 16):
        buf[i, pl.ds(j * 16, 16)] = jnp.zeros((16,), jnp.float32)
      return 0
    lax.fori_loop(0, K, zrow, 0)
    base = s * ROWS_PT
    for kk in range(4):
      pltpu.sync_copy(buf, acc.at[pl.ds(base + kk * K, K)])
    pltpu.sync_copy(buf.at[pl.ds(0, ROWS_PT - 4 * K)],
                    acc.at[pl.ds(base + 4 * K, ROWS_PT - 4 * K)])
    plsc.subcore_barrier()

    def chunk_body(k, _):
      pltpu.sync_copy(xs_hbm.at[c].at[colv.at[k]], buf)
      def scale_row(e, _):
        w = wv[k, e]
        for j in range(DH // 16):
          buf[e, pl.ds(j * 16, 16)] = buf[e, pl.ds(j * 16, 16)] * w
        return 0
      lax.fori_loop(0, K, scale_row, 0)
      pltpu.sync_copy(buf, acc.at[rowv.at[k]], add=True)
      return 0
    lax.fori_loop(0, NCHUNK, chunk_body, 0)
    plsc.subcore_barrier()

    def drain(r0, nrows):
      pltpu.sync_copy(acc.at[pl.ds(r0, nrows)], buf.at[pl.ds(0, nrows)])
      pltpu.sync_copy(buf.at[pl.ds(0, nrows)],
                      out_hbm.at[pl.ds(r0, nrows), pl.ds(c * DH, DH)])
    for kk in range(4):
      drain(base + kk * K, K)
    drain(base + 4 * K, ROWS_PT - 4 * K)

  return agg_kernel(xs, col_r, row_r, ew_r)


def _tc_matmul_bias_relu(agg, W, b2):
  BM = 1000

  def mm_body(a_ref, w_ref, b_ref, o_ref):
    h = lax.dot_general(a_ref[...], w_ref[...],
                        (((1,), (1,)), ((), ())),
                        preferred_element_type=jnp.float32)
    o_ref[...] = jnp.maximum(h + b_ref[...], 0.0)

  return pl.pallas_call(
      mm_body,
      out_shape=jax.ShapeDtypeStruct((N_NODES, D), jnp.float32),
      grid=(N_NODES // BM,),
      in_specs=[
          pl.BlockSpec((BM, D), lambda i: (i, 0)),
          pl.BlockSpec((D, D), lambda i: (0, 0)),
          pl.BlockSpec((1, D), lambda i: (0, 0)),
      ],
      out_specs=pl.BlockSpec((BM, D), lambda i: (i, 0)),
  )(agg, W, b2)


def kernel(x, edge_index, edge_weight, W, b):
  row = edge_index[0].astype(jnp.int32)
  col = edge_index[1].astype(jnp.int32)
  ew = edge_weight.astype(jnp.float32)

  e = row.shape[0]
  pad = E_PAD - e
  row_p = jnp.concatenate([row, jnp.zeros((pad,), jnp.int32)])
  col_p = jnp.concatenate([col, jnp.zeros((pad,), jnp.int32)])
  ew_p = jnp.concatenate([ew, jnp.zeros((pad,), jnp.float32)])

  col_r = col_p.reshape(NS, NCHUNK, K)
  row_r = row_p.reshape(NS, NCHUNK, K)
  ew_r = ew_p.reshape(NS, NCHUNK, K)

  xs = jnp.stack([x[:, :DH], x[:, DH:]])

  agg = _sc_aggregate(xs, col_r, row_r, ew_r)
  return _tc_matmul_bias_relu(agg, W, b[None, :])


# speedup vs baseline: 2.8888x; 2.8888x over previous
import functools

import jax
import jax.numpy as jnp
from jax import lax
from jax.experimental import pallas as pl
from jax.experimental.pallas import tpu as pltpu
from jax.experimental.pallas import tpu_sc as plsc

N_NODES = 10000
D = 256
DH = 128

NC = 2
NS = 16
K = 128

E_PAD = 163840
EPT = E_PAD // NS
NCHUNK = EPT // K
ROWS_PT = N_NODES // NS


def _sc_aggregate(xs, col_r, row_r, ew_r):
  mesh = plsc.VectorSubcoreMesh(core_axis_name="c", subcore_axis_name="s")

  @functools.partial(
      pl.kernel,
      out_type=jax.ShapeDtypeStruct((N_NODES, D), jnp.float32),
      mesh=mesh,
      scratch_types=[
          pltpu.VMEM((NCHUNK, K), jnp.int32),
          pltpu.VMEM((NCHUNK, K), jnp.int32),
          pltpu.VMEM((NCHUNK, K), jnp.float32),
          pltpu.VMEM((K, DH), jnp.float32),
          pltpu.VMEM_SHARED((N_NODES, DH), jnp.float32),
      ],
  )
  def agg_kernel(xs_hbm, col_hbm, row_hbm, ew_hbm, out_hbm,
                 colv, rowv, wv, buf, acc):
    c = lax.axis_index("c")
    s = lax.axis_index("s")

    pltpu.sync_copy(col_hbm.at[s], colv)
    pltpu.sync_copy(row_hbm.at[s], rowv)
    pltpu.sync_copy(ew_hbm.at[s], wv)

    def zrow(i, _):
      for j in range(DH // 16):
        buf[i, pl.ds(j * 16, 16)] = jnp.zeros((16,), jnp.float32)
      return 0
    lax.fori_loop(0, K, zrow, 0)
    base = s * ROWS_PT
    for kk in range(4):
      pltpu.sync_copy(buf, acc.at[pl.ds(base + kk * K, K)])
    pltpu.sync_copy(buf.at[pl.ds(0, ROWS_PT - 4 * K)],
                    acc.at[pl.ds(base + 4 * K, ROWS_PT - 4 * K)])
    plsc.subcore_barrier()

    def chunk_body(k, _):
      pltpu.sync_copy(xs_hbm.at[c].at[colv.at[k]], buf)
      def scale_group(g, _):
        w16 = wv[k, pl.ds(g * 16, 16)]
        e0 = g * 16
        for e in range(16):
          w = w16[e]
          for j in range(DH // 16):
            buf[e0 + e, pl.ds(j * 16, 16)] = buf[e0 + e, pl.ds(j * 16, 16)] * w
        return 0
      lax.fori_loop(0, K // 16, scale_group, 0)
      pltpu.sync_copy(buf, acc.at[rowv.at[k]], add=True)
      return 0
    lax.fori_loop(0, NCHUNK, chunk_body, 0)
    plsc.subcore_barrier()

    nfull = N_NODES // K
    def drain_chunk(t, _):
      cid = s + NS * t
      @pl.when(cid < nfull)
      def _():
        r0 = cid * K
        pltpu.sync_copy(acc.at[pl.ds(r0, K)], buf)
        pltpu.sync_copy(buf, out_hbm.at[pl.ds(r0, K), pl.ds(c * DH, DH)])
      return 0
    lax.fori_loop(0, (nfull + NS - 1) // NS, drain_chunk, 0)
    tail = N_NODES - nfull * K
    @pl.when(s == NS - 1)
    def _():
      pltpu.sync_copy(acc.at[pl.ds(nfull * K, tail)], buf.at[pl.ds(0, tail)])
      pltpu.sync_copy(buf.at[pl.ds(0, tail)],
                      out_hbm.at[pl.ds(nfull * K, tail), pl.ds(c * DH, DH)])

  return agg_kernel(xs, col_r, row_r, ew_r)


def _tc_matmul_bias_relu(agg, W, b2):
  BM = 1000

  def mm_body(a_ref, w_ref, b_ref, o_ref):
    h = lax.dot_general(a_ref[...], w_ref[...],
                        (((1,), (1,)), ((), ())),
                        preferred_element_type=jnp.float32)
    o_ref[...] = jnp.maximum(h + b_ref[...], 0.0)

  return pl.pallas_call(
      mm_body,
      out_shape=jax.ShapeDtypeStruct((N_NODES, D), jnp.float32),
      grid=(N_NODES // BM,),
      in_specs=[
          pl.BlockSpec((BM, D), lambda i: (i, 0)),
          pl.BlockSpec((D, D), lambda i: (0, 0)),
          pl.BlockSpec((1, D), lambda i: (0, 0)),
      ],
      out_specs=pl.BlockSpec((BM, D), lambda i: (i, 0)),
  )(agg, W, b2)


def kernel(x, edge_index, edge_weight, W, b):
  row = edge_index[0].astype(jnp.int32)
  col = edge_index[1].astype(jnp.int32)
  ew = edge_weight.astype(jnp.float32)

  e = row.shape[0]
  pad = E_PAD - e
  row_p = jnp.concatenate([row, jnp.zeros((pad,), jnp.int32)])
  col_p = jnp.concatenate([col, jnp.zeros((pad,), jnp.int32)])
  ew_p = jnp.concatenate([ew, jnp.zeros((pad,), jnp.float32)])

  col_r = col_p.reshape(NS, NCHUNK, K)
  row_r = row_p.reshape(NS, NCHUNK, K)
  ew_r = ew_p.reshape(NS, NCHUNK, K)

  xs = jnp.stack([x[:, :DH], x[:, DH:]])

  agg = _sc_aggregate(xs, col_r, row_r, ew_r)
  return _tc_matmul_bias_relu(agg, W, b[None, :])


# speedup vs baseline: 3.6704x; 1.2706x over previous
import functools

import jax
import jax.numpy as jnp
from jax import lax
from jax.experimental import pallas as pl
from jax.experimental.pallas import tpu as pltpu
from jax.experimental.pallas import tpu_sc as plsc

N_NODES = 10000
D = 256
DH = 128

NC = 2
NS = 16
K = 128

EPT = 10240
E_PAD = EPT * NS
NCHUNK = EPT // K
ROWS_PT = N_NODES // NS
RBITS = 14


def _sc_aggregate(xs, packed_r, ew_r):
  mesh = plsc.VectorSubcoreMesh(core_axis_name="c", subcore_axis_name="s")

  @functools.partial(
      pl.kernel,
      out_type=jax.ShapeDtypeStruct((N_NODES, D), jnp.float32),
      mesh=mesh,
      scratch_types=[
          pltpu.VMEM((NCHUNK, K), jnp.int32),
          pltpu.VMEM((2, K), jnp.float32),
          pltpu.VMEM((2, K), jnp.int32),
          pltpu.VMEM((2, K), jnp.int32),
          pltpu.VMEM((K, DH), jnp.float32),
          pltpu.VMEM((K, DH), jnp.float32),
          pltpu.VMEM_SHARED((N_NODES, DH), jnp.float32),
          pltpu.SemaphoreType.DMA,
          pltpu.SemaphoreType.DMA,
          pltpu.SemaphoreType.DMA,
          pltpu.SemaphoreType.DMA,
          pltpu.SemaphoreType.DMA,
          pltpu.SemaphoreType.DMA,
      ],
  )
  def agg_kernel(xs_hbm, packed_hbm, ew_hbm, out_hbm,
                 pk, wvd, colk, rowk, buf, buf1, acc,
                 gs0, gs1, ss0, ss1, ws0, ws1):
    bufs = (buf, buf1)
    gsem = (gs0, gs1)
    ssem = (ss0, ss1)
    wsem = (ws0, ws1)
    c = lax.axis_index("c")
    s = lax.axis_index("s")

    pltpu.sync_copy(packed_hbm.at[s], pk)

    mask = jnp.full((16,), (1 << RBITS) - 1, jnp.int32)
    def unpack(k, p):
      for g in range(K // 16):
        v = pk[k, pl.ds(g * 16, 16)]
        colk[p, pl.ds(g * 16, 16)] = v & mask
        rowk[p, pl.ds(g * 16, 16)] = lax.shift_right_logical(v, RBITS)

    def zrow(i, _):
      for j in range(DH // 16):
        buf[i, pl.ds(j * 16, 16)] = jnp.zeros((16,), jnp.float32)
      return 0
    lax.fori_loop(0, K, zrow, 0)
    base = s * ROWS_PT
    nz = ROWS_PT // K
    for kk in range(nz):
      pltpu.sync_copy(buf, acc.at[pl.ds(base + kk * K, K)])
    pltpu.sync_copy(buf.at[pl.ds(0, ROWS_PT - nz * K)],
                    acc.at[pl.ds(base + nz * K, ROWS_PT - nz * K)])
    plsc.subcore_barrier()

    def fire_gather(k, p):
      pltpu.async_copy(ew_hbm.at[s].at[k], wvd.at[p], wsem[p])
      pltpu.async_copy(xs_hbm.at[c].at[colk.at[p]], bufs[p], gsem[p])

    def wait_gather(p):
      pltpu.make_async_copy(ew_hbm.at[s].at[0], wvd.at[p], wsem[p]).wait()
      pltpu.make_async_copy(xs_hbm.at[c].at[pl.ds(0, K)], bufs[p],
                            gsem[p]).wait()

    def fire_scatter(k, p):
      pltpu.async_copy(bufs[p], acc.at[rowk.at[p]], ssem[p], add=True)

    def wait_scatter(p):
      pltpu.make_async_copy(bufs[p], acc.at[pl.ds(0, K)], ssem[p]).wait()

    def scale(k, p):
      b = bufs[p]
      def scale_group(g, _):
        w16 = wvd[p, pl.ds(g * 16, 16)]
        e0 = g * 16
        for e in range(16):
          w = w16[e]
          for j in range(DH // 16):
            b[e0 + e, pl.ds(j * 16, 16)] = b[e0 + e, pl.ds(j * 16, 16)] * w
        return 0
      lax.fori_loop(0, K // 16, scale_group, 0)

    unpack(0, 0)
    fire_gather(0, 0)
    def pair_body(kp, _):
      for p in (0, 1):
        k = 2 * kp + p
        q = 1 - p
        @pl.when(k + 1 < NCHUNK)
        def _():
          @pl.when(k >= 1)
          def _():
            wait_scatter(q)
          unpack(k + 1, q)
          fire_gather(k + 1, q)
        wait_gather(p)
        scale(k, p)
        fire_scatter(k, p)
      return 0
    lax.fori_loop(0, NCHUNK // 2, pair_body, 0)
    wait_scatter(0)
    wait_scatter(1)
    plsc.subcore_barrier()

    nfull = N_NODES // K
    def drain_chunk(t, _):
      cid = s + NS * t
      @pl.when(cid < nfull)
      def _():
        r0 = cid * K
        pltpu.sync_copy(acc.at[pl.ds(r0, K)], buf)
        pltpu.sync_copy(buf, out_hbm.at[pl.ds(r0, K), pl.ds(c * DH, DH)])
      return 0
    lax.fori_loop(0, (nfull + NS - 1) // NS, drain_chunk, 0)
    tail = N_NODES - nfull * K
    @pl.when(s == NS - 1)
    def _():
      pltpu.sync_copy(acc.at[pl.ds(nfull * K, tail)], buf.at[pl.ds(0, tail)])
      pltpu.sync_copy(buf.at[pl.ds(0, tail)],
                      out_hbm.at[pl.ds(nfull * K, tail), pl.ds(c * DH, DH)])

  return agg_kernel(xs, packed_r, ew_r)


def _tc_matmul_bias_relu(agg, W, b2):
  BM = 1000

  def mm_body(a_ref, w_ref, b_ref, o_ref):
    h = lax.dot_general(a_ref[...], w_ref[...],
                        (((1,), (1,)), ((), ())),
                        preferred_element_type=jnp.float32)
    o_ref[...] = jnp.maximum(h + b_ref[...], 0.0)

  return pl.pallas_call(
      mm_body,
      out_shape=jax.ShapeDtypeStruct((N_NODES, D), jnp.float32),
      grid=(N_NODES // BM,),
      in_specs=[
          pl.BlockSpec((BM, D), lambda i: (i, 0)),
          pl.BlockSpec((D, D), lambda i: (0, 0)),
          pl.BlockSpec((1, D), lambda i: (0, 0)),
      ],
      out_specs=pl.BlockSpec((BM, D), lambda i: (i, 0)),
  )(agg, W, b2)


def kernel(x, edge_index, edge_weight, W, b):
  row = edge_index[0].astype(jnp.int32)
  col = edge_index[1].astype(jnp.int32)
  ew = edge_weight.astype(jnp.float32)

  e = row.shape[0]
  pad = E_PAD - e
  packed = (row << RBITS) | col
  packed_p = jnp.concatenate([packed, jnp.zeros((pad,), jnp.int32)])
  ew_p = jnp.concatenate([ew, jnp.zeros((pad,), jnp.float32)])

  packed_r = packed_p.reshape(NS, NCHUNK, K)
  ew_r = ew_p.reshape(NS, NCHUNK, K)

  xs = jnp.stack([x[:, :DH], x[:, DH:]])

  agg = _sc_aggregate(xs, packed_r, ew_r)
  return _tc_matmul_bias_relu(agg, W, b[None, :])


# speedup vs baseline: 3.7211x; 1.0138x over previous
import functools

import jax
import jax.numpy as jnp
from jax import lax
from jax.experimental import pallas as pl
from jax.experimental.pallas import tpu as pltpu
from jax.experimental.pallas import tpu_sc as plsc

N_NODES = 10000
D = 256
DH = 128

NC = 2
NS = 16
K = 128

EPT = 10240
E_PAD = EPT * NS
NCHUNK = EPT // K
ROWS_PT = N_NODES // NS
RBITS = 14


def _sc_aggregate(xs, packed_r, ew_r):
  mesh = plsc.VectorSubcoreMesh(core_axis_name="c", subcore_axis_name="s")

  @functools.partial(
      pl.kernel,
      out_type=jax.ShapeDtypeStruct((N_NODES, D), jnp.float32),
      mesh=mesh,
      scratch_types=[
          pltpu.VMEM((NCHUNK, K), jnp.int32),
          pltpu.VMEM((2, K), jnp.float32),
          pltpu.VMEM((2, K), jnp.int32),
          pltpu.VMEM((2, K), jnp.int32),
          pltpu.VMEM((K, DH), jnp.float32),
          pltpu.VMEM((K, DH), jnp.float32),
          pltpu.VMEM_SHARED((N_NODES, DH), jnp.float32),
          pltpu.SemaphoreType.DMA,
          pltpu.SemaphoreType.DMA,
          pltpu.SemaphoreType.DMA,
          pltpu.SemaphoreType.DMA,
          pltpu.SemaphoreType.DMA,
          pltpu.SemaphoreType.DMA,
      ],
  )
  def agg_kernel(xs_hbm, packed_hbm, ew_hbm, out_hbm,
                 pk, wvd, colk, rowk, buf, buf1, acc,
                 gs0, gs1, ss0, ss1, ws0, ws1):
    bufs = (buf, buf1)
    gsem = (gs0, gs1)
    ssem = (ss0, ss1)
    wsem = (ws0, ws1)
    c = lax.axis_index("c")
    s = lax.axis_index("s")

    pltpu.sync_copy(packed_hbm.at[s], pk)

    mask = jnp.full((16,), (1 << RBITS) - 1, jnp.int32)
    def unpack(k, p):
      for g in range(K // 16):
        v = pk[k, pl.ds(g * 16, 16)]
        colk[p, pl.ds(g * 16, 16)] = v & mask
        rowk[p, pl.ds(g * 16, 16)] = lax.shift_right_logical(v, RBITS)

    def zrow(i, _):
      for j in range(DH // 16):
        buf[i, pl.ds(j * 16, 16)] = jnp.zeros((16,), jnp.float32)
      return 0
    lax.fori_loop(0, K, zrow, 0)
    base = s * ROWS_PT
    nz = ROWS_PT // K
    for kk in range(nz):
      pltpu.sync_copy(buf, acc.at[pl.ds(base + kk * K, K)])
    pltpu.sync_copy(buf.at[pl.ds(0, ROWS_PT - nz * K)],
                    acc.at[pl.ds(base + nz * K, ROWS_PT - nz * K)])
    plsc.subcore_barrier()

    def fire_gather(k, p):
      pltpu.async_copy(ew_hbm.at[s].at[k], wvd.at[p], wsem[p])
      pltpu.async_copy(xs_hbm.at[c].at[colk.at[p]], bufs[p], gsem[p])

    def wait_gather(p):
      pltpu.make_async_copy(ew_hbm.at[s].at[0], wvd.at[p], wsem[p]).wait()
      pltpu.make_async_copy(xs_hbm.at[c].at[pl.ds(0, K)], bufs[p],
                            gsem[p]).wait()

    def fire_scatter(k, p):
      pltpu.async_copy(bufs[p], acc.at[rowk.at[p]], ssem[p], add=True)

    def wait_scatter(p):
      pltpu.make_async_copy(bufs[p], acc.at[pl.ds(0, K)], ssem[p]).wait()

    def scale(k, p):
      b = bufs[p]
      for g in range(K // 16):
        w16 = wvd[p, pl.ds(g * 16, 16)]
        for e in range(16):
          w = w16[e]
          r = g * 16 + e
          for j in range(DH // 16):
            b[r, pl.ds(j * 16, 16)] = b[r, pl.ds(j * 16, 16)] * w

    unpack(0, 0)
    fire_gather(0, 0)
    def pair_body(kp, _):
      for p in (0, 1):
        k = 2 * kp + p
        q = 1 - p
        @pl.when(k + 1 < NCHUNK)
        def _():
          @pl.when(k >= 1)
          def _():
            wait_scatter(q)
          unpack(k + 1, q)
          fire_gather(k + 1, q)
        wait_gather(p)
        scale(k, p)
        fire_scatter(k, p)
      return 0
    lax.fori_loop(0, NCHUNK // 2, pair_body, 0)
    wait_scatter(0)
    wait_scatter(1)
    plsc.subcore_barrier()

    nfull = N_NODES // K
    def drain_chunk(t, _):
      cid = s + NS * t
      @pl.when(cid < nfull)
      def _():
        r0 = cid * K
        pltpu.sync_copy(acc.at[pl.ds(r0, K)], buf)
        pltpu.sync_copy(buf, out_hbm.at[pl.ds(r0, K), pl.ds(c * DH, DH)])
      return 0
    lax.fori_loop(0, (nfull + NS - 1) // NS, drain_chunk, 0)
    tail = N_NODES - nfull * K
    @pl.when(s == NS - 1)
    def _():
      pltpu.sync_copy(acc.at[pl.ds(nfull * K, tail)], buf.at[pl.ds(0, tail)])
      pltpu.sync_copy(buf.at[pl.ds(0, tail)],
                      out_hbm.at[pl.ds(nfull * K, tail), pl.ds(c * DH, DH)])

  return agg_kernel(xs, packed_r, ew_r)


def _tc_matmul_bias_relu(agg, W, b2):
  BM = 1000

  def mm_body(a_ref, w_ref, b_ref, o_ref):
    h = lax.dot_general(a_ref[...], w_ref[...],
                        (((1,), (1,)), ((), ())),
                        preferred_element_type=jnp.float32)
    o_ref[...] = jnp.maximum(h + b_ref[...], 0.0)

  return pl.pallas_call(
      mm_body,
      out_shape=jax.ShapeDtypeStruct((N_NODES, D), jnp.float32),
      grid=(N_NODES // BM,),
      in_specs=[
          pl.BlockSpec((BM, D), lambda i: (i, 0)),
          pl.BlockSpec((D, D), lambda i: (0, 0)),
          pl.BlockSpec((1, D), lambda i: (0, 0)),
      ],
      out_specs=pl.BlockSpec((BM, D), lambda i: (i, 0)),
  )(agg, W, b2)


def kernel(x, edge_index, edge_weight, W, b):
  row = edge_index[0].astype(jnp.int32)
  col = edge_index[1].astype(jnp.int32)
  ew = edge_weight.astype(jnp.float32)

  e = row.shape[0]
  pad = E_PAD - e
  packed = (row << RBITS) | col
  packed_p = jnp.concatenate([packed, jnp.zeros((pad,), jnp.int32)])
  ew_p = jnp.concatenate([ew, jnp.zeros((pad,), jnp.float32)])

  packed_r = packed_p.reshape(NS, NCHUNK, K)
  ew_r = ew_p.reshape(NS, NCHUNK, K)

  xs = jnp.stack([x[:, :DH], x[:, DH:]])

  agg = _sc_aggregate(xs, packed_r, ew_r)
  return _tc_matmul_bias_relu(agg, W, b[None, :])
